# baseline (speedup 1.0000x reference)
import functools

import jax
import jax.numpy as jnp
from jax.experimental import pallas as pl
from jax.experimental.pallas import tpu as pltpu

INPUT_DIM = 768
OUTPUT_DIM = 768
HIDDEN = 128
NUM_EXPERTS = 64
TOP_K = 2
N_TOKENS = 4096


def _moe_body(x_ref, W1_ref, b1_ref, W2_ref, b2_ref, wg_ref,
              out_ref, aux_ref, g_ref):
    e = pl.program_id(0)

    @pl.when(e == 0)
    def _gating():
        x = x_ref[...]
        logits = jnp.dot(x, wg_ref[...], preferred_element_type=jnp.float32)
        m = jnp.max(logits, axis=1, keepdims=True)
        ex = jnp.exp(logits - m)
        gates = ex / jnp.sum(ex, axis=1, keepdims=True)
        importance = jnp.mean(gates, axis=0)
        tgt = 1.0 / NUM_EXPERTS
        aux = jnp.sum(tgt * (jnp.log(tgt) - jnp.log(importance)))
        aux_ref[...] = aux.reshape(1, 1)
        eids = jax.lax.broadcasted_iota(jnp.int32, logits.shape, 1)
        m0 = jnp.max(logits, axis=1, keepdims=True)
        is0 = logits == m0
        idx0 = jnp.min(jnp.where(is0, eids, NUM_EXPERTS), axis=1, keepdims=True)
        neg = jnp.float32(-jnp.inf)
        logits1 = jnp.where(eids == idx0, neg, logits)
        m1 = jnp.max(logits1, axis=1, keepdims=True)
        is1 = logits1 == m1
        idx1 = jnp.min(jnp.where(is1, eids, NUM_EXPERTS), axis=1, keepdims=True)
        w0 = 1.0 / (1.0 + jnp.exp(m1 - m0))
        w1 = 1.0 - w0
        g_ref[...] = jnp.where(eids == idx0, w0,
                               jnp.where(eids == idx1, w1, 0.0))
        out_ref[...] = jnp.zeros_like(out_ref)

    x = x_ref[...].astype(jnp.bfloat16)
    h = jnp.dot(x, W1_ref[0], preferred_element_type=jnp.float32)
    h = jnp.maximum(h + b1_ref[0], 0.0).astype(jnp.bfloat16)
    y = jnp.dot(h, W2_ref[0], preferred_element_type=jnp.float32)
    y = y + b2_ref[0]
    g = g_ref[...]
    lane = jax.lax.broadcasted_iota(jnp.int32, g.shape, 1)
    g_col = jnp.sum(jnp.where(lane == e, g, 0.0), axis=1, keepdims=True)
    out_ref[...] += g_col * y


@jax.jit
def kernel(x, W1, b1, W2, b2, w_gate):
    out, aux = pl.pallas_call(
        _moe_body,
        grid=(NUM_EXPERTS,),
        in_specs=[
            pl.BlockSpec((N_TOKENS, INPUT_DIM), lambda e: (0, 0)),
            pl.BlockSpec((1, INPUT_DIM, HIDDEN), lambda e: (e, 0, 0)),
            pl.BlockSpec((1, 1, HIDDEN), lambda e: (e, 0, 0)),
            pl.BlockSpec((1, HIDDEN, OUTPUT_DIM), lambda e: (e, 0, 0)),
            pl.BlockSpec((1, 1, OUTPUT_DIM), lambda e: (e, 0, 0)),
            pl.BlockSpec((INPUT_DIM, NUM_EXPERTS), lambda e: (0, 0)),
        ],
        out_specs=[
            pl.BlockSpec((N_TOKENS, OUTPUT_DIM), lambda e: (0, 0)),
            pl.BlockSpec((1, 1), lambda e: (0, 0)),
        ],
        out_shape=[
            jax.ShapeDtypeStruct((N_TOKENS, OUTPUT_DIM), jnp.float32),
            jax.ShapeDtypeStruct((1, 1), jnp.float32),
        ],
        scratch_shapes=[pltpu.VMEM((N_TOKENS, NUM_EXPERTS), jnp.float32)],
    )(x, W1.astype(jnp.bfloat16), b1.reshape(NUM_EXPERTS, 1, HIDDEN),
      W2.astype(jnp.bfloat16), b2.reshape(NUM_EXPERTS, 1, OUTPUT_DIM), w_gate)
    return out, aux[0, 0]


# speedup vs baseline: 2.2417x; 2.2417x over previous
import functools

import jax
import jax.numpy as jnp
from jax import lax
from jax.experimental import pallas as pl
from jax.experimental.pallas import tpu as pltpu
from jax.experimental.pallas import tpu_sc as plsc

INPUT_DIM = 768
OUTPUT_DIM = 768
HIDDEN = 128
NUM_EXPERTS = 64
N_TOKENS = 4096

TILE = 256
NT = (N_TOKENS * 2) // TILE + NUM_EXPERTS
P = NT * TILE
CBLK = 256

SC_CORES = 2
SC_SUBCORES = 16
SC_WORKERS = SC_CORES * SC_SUBCORES
TOK_W = N_TOKENS // SC_WORKERS


def _route_body(x_ref, wg_ref,
                aux_ref, w0_ref, w1_ref, pos_ref, te_ref, fetch_ref, used_ref,
                a_s, r_s):
    x = x_ref[...]
    logits = jnp.dot(x, wg_ref[...], preferred_element_type=jnp.float32)
    m = jnp.max(logits, axis=1, keepdims=True)
    ex = jnp.exp(logits - m)
    gates = ex / jnp.sum(ex, axis=1, keepdims=True)
    importance = jnp.mean(gates, axis=0)
    tgt = 1.0 / NUM_EXPERTS
    aux_ref[...] = jnp.sum(tgt * (jnp.log(tgt) - jnp.log(importance))).reshape(1, 1)
    eids = lax.broadcasted_iota(jnp.int32, logits.shape, 1)
    is0 = logits == m
    idx0 = jnp.min(jnp.where(is0, eids, NUM_EXPERTS), axis=1, keepdims=True)
    logits1 = jnp.where(eids == idx0, -jnp.inf, logits)
    m1 = jnp.max(logits1, axis=1, keepdims=True)
    is1 = logits1 == m1
    idx1 = jnp.min(jnp.where(is1, eids, NUM_EXPERTS), axis=1, keepdims=True)
    w0 = 1.0 / (1.0 + jnp.exp(m1 - m))
    w0_ref[...] = w0
    w1_ref[...] = 1.0 - w0
    a_s[...] = jnp.where((eids == idx0) | (eids == idx1), 1.0, 0.0)
    ri = lax.broadcasted_iota(jnp.int32, (CBLK, CBLK), 0)
    ci = lax.broadcasted_iota(jnp.int32, (CBLK, CBLK), 1)
    ltri = jnp.where(ci < ri, 1.0, 0.0)

    def blk(j, base):
        off = pl.multiple_of(j * CBLK, CBLK)
        ab = a_s[pl.ds(off, CBLK), :]
        r_s[pl.ds(off, CBLK), :] = (
            jnp.dot(ltri, ab, preferred_element_type=jnp.float32) + base)
        return base + jnp.sum(ab, axis=0, keepdims=True)

    counts = lax.fori_loop(0, N_TOKENS // CBLK, blk,
                           jnp.zeros((1, NUM_EXPERTS), jnp.float32))
    pci = ((counts.astype(jnp.int32) + (TILE - 1)) // TILE) * TILE
    pc = pci.astype(jnp.float32)
    fe = lax.broadcasted_iota(jnp.int32, (NUM_EXPERTS, NUM_EXPERTS), 0)
    ee = lax.broadcasted_iota(jnp.int32, (NUM_EXPERTS, NUM_EXPERTS), 1)
    excl = jnp.where(fe < ee, 1.0, 0.0)
    astart = jnp.dot(pc, excl, preferred_element_type=jnp.float32)
    r = r_s[...]
    asb = jnp.broadcast_to(astart, (N_TOKENS, NUM_EXPERTS))
    pos0 = jnp.sum(jnp.where(eids == idx0, r + asb, 0.0), axis=1, keepdims=True)
    pos1 = jnp.sum(jnp.where(eids == idx1, r + asb, 0.0), axis=1, keepdims=True)
    pos_ref[...] = jnp.concatenate([pos0, pos1], axis=1).astype(jnp.int32)
    ident = jnp.where(
        lax.broadcasted_iota(jnp.int32, (NUM_EXPERTS, NUM_EXPERTS), 0)
        == lax.broadcasted_iota(jnp.int32, (NUM_EXPERTS, NUM_EXPERTS), 1),
        1.0, 0.0)
    astart_col = lax.dot_general(ident, astart, (((1,), (1,)), ((), ())),
                                 preferred_element_type=jnp.float32)
    t_iota = lax.broadcasted_iota(jnp.int32, (1, NT), 1)
    tstart = (t_iota * TILE).astype(jnp.float32)
    te = jnp.sum(jnp.where(astart_col <= tstart, 1.0, 0.0),
                 axis=0, keepdims=True) - 1.0
    te_ref[...] = te.astype(jnp.int32)
    total = jnp.sum(pc)
    used = jnp.where(tstart < total, 1, 0)
    used_ref[...] = used
    fetch_ref[...] = t_iota * used


def _route(x, w_gate):
    return pl.pallas_call(
        _route_body,
        out_shape=[
            jax.ShapeDtypeStruct((1, 1), jnp.float32),
            jax.ShapeDtypeStruct((N_TOKENS, 1), jnp.float32),
            jax.ShapeDtypeStruct((N_TOKENS, 1), jnp.float32),
            jax.ShapeDtypeStruct((N_TOKENS, 2), jnp.int32),
            jax.ShapeDtypeStruct((1, NT), jnp.int32),
            jax.ShapeDtypeStruct((1, NT), jnp.int32),
            jax.ShapeDtypeStruct((1, NT), jnp.int32),
        ],
        scratch_shapes=[
            pltpu.VMEM((N_TOKENS, NUM_EXPERTS), jnp.float32),
            pltpu.VMEM((N_TOKENS, NUM_EXPERTS), jnp.float32),
        ],
    )(x, w_gate)


def _sc_scatter(x, pos_sc):
    mesh = plsc.VectorSubcoreMesh(core_axis_name="c", subcore_axis_name="s")

    @functools.partial(
        pl.kernel, mesh=mesh,
        out_type=jax.ShapeDtypeStruct((P, INPUT_DIM), jnp.float32),
        scratch_types=[
            pltpu.VMEM((TOK_W,), jnp.int32),
            pltpu.VMEM((TOK_W,), jnp.int32),
            pltpu.VMEM((TOK_W, INPUT_DIM), jnp.float32),
            pltpu.SemaphoreType.DMA,
        ],
    )
    def k(x_hbm, pos_hbm, xs_hbm, idx0_v, idx1_v, rows_v, sem):
        wid = lax.axis_index("s") * SC_CORES + lax.axis_index("c")
        base = wid * TOK_W
        pltpu.sync_copy(x_hbm.at[pl.ds(base, TOK_W)], rows_v)
        pltpu.sync_copy(pos_hbm.at[0, wid], idx0_v)
        pltpu.sync_copy(pos_hbm.at[1, wid], idx1_v)
        pltpu.async_copy(rows_v, xs_hbm.at[idx0_v], sem).wait()
        pltpu.async_copy(rows_v, xs_hbm.at[idx1_v], sem).wait()

    return k(x, pos_sc)


def _group_body(te_ref, fetch_ref, used_ref,
                xs_ref, W1_ref, b1_ref, W2_ref, b2_ref, ys_ref):
    t = pl.program_id(0)

    @pl.when(used_ref[t] == 1)
    def _():
        h = jnp.dot(xs_ref[...], W1_ref[0], preferred_element_type=jnp.float32)
        h = jnp.maximum(h + b1_ref[0], 0.0)
        y = jnp.dot(h, W2_ref[0], preferred_element_type=jnp.float32)
        ys_ref[...] = y + b2_ref[0]


def _grouped_mlp(te, fetch, used, xs, W1, b1r, W2, b2r):
    spec = pltpu.PrefetchScalarGridSpec(
        num_scalar_prefetch=3,
        grid=(NT,),
        in_specs=[
            pl.BlockSpec((TILE, INPUT_DIM), lambda t, te, f, u: (f[t], 0)),
            pl.BlockSpec((1, INPUT_DIM, HIDDEN), lambda t, te, f, u: (te[t], 0, 0)),
            pl.BlockSpec((1, 1, HIDDEN), lambda t, te, f, u: (te[t], 0, 0)),
            pl.BlockSpec((1, HIDDEN, OUTPUT_DIM), lambda t, te, f, u: (te[t], 0, 0)),
            pl.BlockSpec((1, 1, OUTPUT_DIM), lambda t, te, f, u: (te[t], 0, 0)),
        ],
        out_specs=pl.BlockSpec((TILE, OUTPUT_DIM), lambda t, te, f, u: (t, 0)),
    )
    return pl.pallas_call(
        _group_body,
        grid_spec=spec,
        out_shape=jax.ShapeDtypeStruct((P, OUTPUT_DIM), jnp.float32),
    )(te, fetch, used, xs, W1, b1r, W2, b2r)


def _sc_gather(ys, pos_sc):
    mesh = plsc.VectorSubcoreMesh(core_axis_name="c", subcore_axis_name="s")

    @functools.partial(
        pl.kernel, mesh=mesh,
        out_type=[
            jax.ShapeDtypeStruct((N_TOKENS, OUTPUT_DIM), jnp.float32),
            jax.ShapeDtypeStruct((N_TOKENS, OUTPUT_DIM), jnp.float32),
        ],
        scratch_types=[
            pltpu.VMEM((TOK_W,), jnp.int32),
            pltpu.VMEM((TOK_W, OUTPUT_DIM), jnp.float32),
            pltpu.SemaphoreType.DMA,
        ],
    )
    def k(ys_hbm, pos_hbm, g0_hbm, g1_hbm, idx_v, rows_v, sem):
        wid = lax.axis_index("s") * SC_CORES + lax.axis_index("c")
        base = wid * TOK_W
        pltpu.sync_copy(pos_hbm.at[0, wid], idx_v)
        pltpu.async_copy(ys_hbm.at[idx_v], rows_v, sem).wait()
        pltpu.sync_copy(rows_v, g0_hbm.at[pl.ds(base, TOK_W)])
        pltpu.sync_copy(pos_hbm.at[1, wid], idx_v)
        pltpu.async_copy(ys_hbm.at[idx_v], rows_v, sem).wait()
        pltpu.sync_copy(rows_v, g1_hbm.at[pl.ds(base, TOK_W)])

    return k(ys, pos_sc)


def _combine_body(g0_ref, g1_ref, w0_ref, w1_ref, out_ref):
    out_ref[...] = w0_ref[...] * g0_ref[...] + w1_ref[...] * g1_ref[...]


def _combine(g0, g1, w0, w1):
    blk = 512
    return pl.pallas_call(
        _combine_body,
        grid=(N_TOKENS // blk,),
        in_specs=[
            pl.BlockSpec((blk, OUTPUT_DIM), lambda i: (i, 0)),
            pl.BlockSpec((blk, OUTPUT_DIM), lambda i: (i, 0)),
            pl.BlockSpec((blk, 1), lambda i: (i, 0)),
            pl.BlockSpec((blk, 1), lambda i: (i, 0)),
        ],
        out_specs=pl.BlockSpec((blk, OUTPUT_DIM), lambda i: (i, 0)),
        out_shape=jax.ShapeDtypeStruct((N_TOKENS, OUTPUT_DIM), jnp.float32),
    )(g0, g1, w0, w1)


@jax.jit
def kernel(x, W1, b1, W2, b2, w_gate):
    aux, w0, w1, pos, te, fetch, used = _route(x, w_gate)
    pos_sc = pos.T.reshape(2, SC_WORKERS, TOK_W)
    xs = _sc_scatter(x, pos_sc)
    ys = _grouped_mlp(te.reshape(NT), fetch.reshape(NT), used.reshape(NT),
                      xs, W1, b1.reshape(NUM_EXPERTS, 1, HIDDEN),
                      W2, b2.reshape(NUM_EXPERTS, 1, OUTPUT_DIM))
    g0, g1 = _sc_gather(ys, pos_sc)
    out = _combine(g0, g1, w0, w1)
    return out, aux[0, 0]
